# Initial kernel scaffold; baseline (speedup 1.0000x reference)
import jax
import jax.numpy as jnp
from jax.experimental import pallas as pl


def kernel(ref_bxyz, query_bxyz):
    raise NotImplementedError("write your pallas kernel here")


# speedup vs baseline: 2.5706x; 2.5706x over previous
import functools

import jax
import jax.numpy as jnp
from jax import lax
from jax.experimental import pallas as pl
from jax.experimental.pallas import tpu as pltpu
from jax.experimental.pallas import tpu_sc as plsc

_RADIUS2 = 1.0
_K = 32
_N_REF = 30000
_N_QUERY = 30000
_NW = 32
_QCHUNK = 944
_NQ_PAD = _NW * _QCHUNK

_INF = float("inf")


def _merge_top32(a0d, a0i, a1d, a1i, pd, pi):
  rpd = lax.rev(pd, (0,))
  rpi = lax.rev(pi, (0,))
  m = a1d <= rpd
  ld = jnp.where(m, a1d, rpd)
  li = jnp.where(m, a1i, rpi)
  ld, li = plsc.sort_key_val(ld, li)
  rld = lax.rev(ld, (0,))
  rli = lax.rev(li, (0,))
  m2 = a0d <= rld
  b0d = jnp.where(m2, a0d, rld)
  b0i = jnp.where(m2, a0i, rli)
  b1d = jnp.where(m2, rld, a0d)
  b1i = jnp.where(m2, rli, a0i)
  b0d, b0i = plsc.sort_key_val(b0d, b0i)
  b1d, b1i = plsc.sort_key_val(b1d, b1i)
  return b0d, b0i, b1d, b1i


def _radius_body(ref_hbm, q_hbm, out_hbm, refs_v, q_v, row_v):
  cid = lax.axis_index("c")
  sid = lax.axis_index("s")
  wid = sid * 2 + cid
  qbase = wid * _QCHUNK

  pltpu.sync_copy(ref_hbm, refs_v)
  pltpu.sync_copy(q_hbm.at[pl.ds(qbase * 4, _QCHUNK * 4)],
                  q_v.at[pl.ds(0, _QCHUNK * 4)])

  lanes = lax.iota(jnp.int32, 16)
  n_groups = _N_REF // 16

  def per_query(j, _):
    qvec = q_v[pl.ds(j * 4, 16)]
    qb = qvec[0]
    qx = qvec[1]
    qy = qvec[2]
    qz = qvec[3]

    def per_group(g, carry):
      a0d, a0i, a1d, a1i = carry
      rows = g * 16 + lanes
      flat = rows * 4
      rb = plsc.load_gather(refs_v, [flat])
      rx = plsc.load_gather(refs_v, [flat + 1])
      ry = plsc.load_gather(refs_v, [flat + 2])
      rz = plsc.load_gather(refs_v, [flat + 3])
      dx = rx - qx
      dy = ry - qy
      dz = rz - qz
      d2 = dx * dx + dy * dy + dz * dz
      m = (rb == qb) & (d2 <= _RADIUS2)

      def do_merge(args):
        a0d, a0i, a1d, a1i = args
        pd = jnp.where(m, d2, _INF)
        pd, pi = plsc.sort_key_val(pd, rows)
        return _merge_top32(a0d, a0i, a1d, a1i, pd, pi)

      return lax.cond(jnp.any(m), do_merge, lambda args: args,
                      (a0d, a0i, a1d, a1i))

    inf16 = jnp.full((16,), _INF, jnp.float32)
    neg16 = jnp.full((16,), -1, jnp.int32)
    a0d, a0i, a1d, a1i = lax.fori_loop(
        0, n_groups, per_group, (inf16, neg16, inf16, neg16))

    o0 = jnp.where(a0d <= _RADIUS2, a0i, -1)
    o1 = jnp.where(a1d <= _RADIUS2, a1i, -1)
    row_v[0:16] = o0
    row_v[16:32] = o1
    pltpu.sync_copy(row_v, out_hbm.at[pl.ds((qbase + j) * _K, _K)])
    return 0

  lax.fori_loop(0, _QCHUNK, per_query, 0)


def kernel(ref_bxyz, query_bxyz):
  q_pad = jnp.pad(query_bxyz, ((0, _NQ_PAD - _N_QUERY), (0, 0)))
  mesh = plsc.VectorSubcoreMesh(
      core_axis_name="c", subcore_axis_name="s", num_cores=2, num_subcores=16)
  nbr = pl.kernel(
      _radius_body,
      out_type=jax.ShapeDtypeStruct((_NQ_PAD * _K,), jnp.int32),
      mesh=mesh,
      compiler_params=pltpu.CompilerParams(needs_layout_passes=False),
      scratch_types=[
          pltpu.VMEM((_N_REF * 4,), jnp.float32),
          pltpu.VMEM((_QCHUNK * 4 + 16,), jnp.float32),
          pltpu.VMEM((_K,), jnp.int32),
      ],
  )(ref_bxyz.reshape(-1), q_pad.reshape(-1))
  ref_idx = nbr[: _N_QUERY * _K]
  q_idx = jnp.repeat(jnp.arange(_N_QUERY, dtype=jnp.int32), _K)
  return jnp.stack([ref_idx, q_idx])


# speedup vs baseline: 50.5209x; 19.6533x over previous
import functools

import jax
import jax.numpy as jnp
from jax import lax
from jax.experimental import pallas as pl
from jax.experimental.pallas import tpu as pltpu
from jax.experimental.pallas import tpu_sc as plsc

_RADIUS2 = 1.0
_K = 32
_N_REF = 30000
_N_QUERY = 30000
_NW = 32
_CHUNK = 944
_N_PAD = _NW * _CHUNK
_NCELL = 4000
_NCELL_PAD = 4016
_SORT_PAD = _N_PAD + 128
_GROUPS = _CHUNK // 16

_INF = float("inf")


def _mesh():
  return plsc.VectorSubcoreMesh(
      core_axis_name="c", subcore_axis_name="s", num_cores=2, num_subcores=16)


def _wid():
  return lax.axis_index("s") * 2 + lax.axis_index("c")


def _cell_of(rb, rx, ry, rz, valid):
  bi = rb.astype(jnp.int32)
  fx = jnp.clip(rx.astype(jnp.int32), 0, 9)
  fy = jnp.clip(ry.astype(jnp.int32), 0, 9)
  fz = jnp.clip(rz.astype(jnp.int32), 0, 9)
  cell = ((bi * 10 + fx) * 10 + fy) * 10 + fz
  return jnp.where(valid, cell, _NCELL)


def _hist_body(ref_hbm, cells_hbm, hist_hbm, refs_v, hist_v, cells_v):
  wid = _wid()
  base = wid * _CHUNK
  pltpu.sync_copy(ref_hbm.at[pl.ds(base * 4, _CHUNK * 4)],
                  refs_v.at[pl.ds(0, _CHUNK * 4)])
  lanes = lax.iota(jnp.int32, 16)
  zeros = jnp.zeros((16,), jnp.int32)
  ones = jnp.ones((16,), jnp.int32)

  def zero_hist(g, _):
    hist_v[pl.ds(g * 16, 16)] = zeros
    return 0
  lax.fori_loop(0, _NCELL_PAD // 16, zero_hist, 0)

  def per_group(g, _):
    flat = (g * 16 + lanes) * 4
    rb = plsc.load_gather(refs_v, [flat])
    rx = plsc.load_gather(refs_v, [flat + 1])
    ry = plsc.load_gather(refs_v, [flat + 2])
    rz = plsc.load_gather(refs_v, [flat + 3])
    valid = (base + g * 16 + lanes) < _N_REF
    cell = _cell_of(rb, rx, ry, rz, valid)
    cells_v[pl.ds(g * 16, 16)] = cell
    return 0
  lax.fori_loop(0, _GROUPS, per_group, 0)

  def per_elem(i, _):
    c = cells_v[pl.ds(i, 16)][0]
    v = hist_v[pl.ds(c, 16)]
    hist_v[pl.ds(c, 16)] = v + jnp.where(lanes == 0, 1, 0)
    return 0
  lax.fori_loop(0, _CHUNK, per_elem, 0)

  pltpu.sync_copy(cells_v.at[pl.ds(0, _CHUNK)], cells_hbm.at[pl.ds(base, _CHUNK)])
  pltpu.sync_copy(hist_v.at[pl.ds(0, _NCELL_PAD)],
                  hist_hbm.at[pl.ds(wid * _NCELL_PAD, _NCELL_PAD)])


def _scatter_body(ref_hbm, cells_hbm, hist_hbm,
                  sx_hbm, sy_hbm, sz_hbm, si_hbm, start_hbm,
                  refs_v, cells_v, tmp_v, pfx_v, tot_v, start_v,
                  stgx_v, stgy_v, stgz_v, stgi_v, dest_v, dest2_v, sem):
  wid = _wid()
  base = wid * _CHUNK
  pltpu.sync_copy(ref_hbm.at[pl.ds(base * 4, _CHUNK * 4)],
                  refs_v.at[pl.ds(0, _CHUNK * 4)])
  pltpu.sync_copy(cells_hbm.at[pl.ds(base, _CHUNK)],
                  cells_v.at[pl.ds(0, _CHUNK)])
  lanes = lax.iota(jnp.int32, 16)
  zeros = jnp.zeros((16,), jnp.int32)
  ngrp = _NCELL_PAD // 16

  def zero2(g, _):
    pfx_v[pl.ds(g * 16, 16)] = zeros
    tot_v[pl.ds(g * 16, 16)] = zeros
    return 0
  lax.fori_loop(0, ngrp, zero2, 0)

  def per_worker(t, _):
    pltpu.sync_copy(hist_hbm.at[pl.ds(t * _NCELL_PAD, _NCELL_PAD)], tmp_v)

    def acc(g, _):
      h = tmp_v[pl.ds(g * 16, 16)]
      tot_v[pl.ds(g * 16, 16)] = tot_v[pl.ds(g * 16, 16)] + h

      @pl.when(t < wid)
      def _():
        pfx_v[pl.ds(g * 16, 16)] = pfx_v[pl.ds(g * 16, 16)] + h
      return 0
    lax.fori_loop(0, ngrp, acc, 0)
    return 0
  lax.fori_loop(0, _NW, per_worker, 0)

  def scan(g, carry):
    v = tot_v[pl.ds(g * 16, 16)]
    s = plsc.cumsum(v)
    excl = s - v + carry
    start_v[pl.ds(g * 16, 16)] = excl
    pfx_v[pl.ds(g * 16, 16)] = pfx_v[pl.ds(g * 16, 16)] + excl
    return carry + s[15]
  lax.fori_loop(0, ngrp, scan, jnp.int32(0))

  @pl.when(wid == 0)
  def _():
    pltpu.sync_copy(start_v.at[pl.ds(0, _NCELL_PAD)],
                    start_hbm.at[pl.ds(0, _NCELL_PAD)])

  def dflt(g, _):
    dest_v[pl.ds(g * 16, 16)] = _N_PAD + ((g * 16 + lanes) % 128)
    return 0
  lax.fori_loop(0, 1024 // 16, dflt, 0)

  def per_elem(i, _):
    c = cells_v[pl.ds(i, 16)][0]
    v = pfx_v[pl.ds(c, 16)]
    d = v[0]
    pfx_v[pl.ds(c, 16)] = v + jnp.where(lanes == 0, 1, 0)
    dv = dest_v[pl.ds(i, 16)]
    dest_v[pl.ds(i, 16)] = jnp.where(lanes == 0, d, dv)
    return 0
  lax.fori_loop(0, _CHUNK, per_elem, 0)

  def stage(g, _):
    flat = (g * 16 + lanes) * 4
    rows = g * 16 + lanes
    sl = pl.ds(g * 16, 16)
    rb = plsc.load_gather(refs_v, [flat])
    stgx_v[sl] = plsc.load_gather(refs_v, [flat + 1])
    stgy_v[sl] = plsc.load_gather(refs_v, [flat + 2])
    stgz_v[sl] = plsc.load_gather(refs_v, [flat + 3])
    stgi_v[sl] = (base + rows) | (rb.astype(jnp.int32) << 20)
    return 0
  lax.fori_loop(0, _GROUPS, stage, 0)

  def repack(k, _):
    j = k // 8
    g = k % 8
    dest2_v[j, pl.ds(g * 16, 16)] = dest_v[pl.ds(j * 128 + g * 16, 16)]
    return 0
  lax.fori_loop(0, 64, repack, 0)

  for j in range(8):
    sl = pl.ds(j * 128, 128)
    idx = dest2_v.at[j]
    pltpu.async_copy(stgx_v.at[sl], sx_hbm.at[idx], sem).wait()
    pltpu.async_copy(stgy_v.at[sl], sy_hbm.at[idx], sem).wait()
    pltpu.async_copy(stgz_v.at[sl], sz_hbm.at[idx], sem).wait()
    pltpu.async_copy(stgi_v.at[sl], si_hbm.at[idx], sem).wait()


def _merge_top32(a0d, a0i, a1d, a1i, pd, pi):
  rpd = lax.rev(pd, (0,))
  rpi = lax.rev(pi, (0,))
  m = a1d <= rpd
  ld = jnp.where(m, a1d, rpd)
  li = jnp.where(m, a1i, rpi)
  ld, li = plsc.sort_key_val(ld, li)
  rld = lax.rev(ld, (0,))
  rli = lax.rev(li, (0,))
  m2 = a0d <= rld
  b0d = jnp.where(m2, a0d, rld)
  b0i = jnp.where(m2, a0i, rli)
  b1d = jnp.where(m2, rld, a0d)
  b1i = jnp.where(m2, rli, a0i)
  b0d, b0i = plsc.sort_key_val(b0d, b0i)
  b1d, b1i = plsc.sort_key_val(b1d, b1i)
  return b0d, b0i, b1d, b1i


_CAP = 416
_NSG = _SORT_PAD // 16


def _query_body(sx_hbm, sy_hbm, sz_hbm, si_hbm, start_hbm, q_hbm, out_hbm,
                sx_v, sy_v, sz_v, si_v, start_v, q_v, row_v, pend_d, pend_i):
  wid = _wid()
  qbase = wid * _CHUNK
  pltpu.sync_copy(sx_hbm, sx_v.at[pl.ds(0, _SORT_PAD)])
  pltpu.sync_copy(sy_hbm, sy_v.at[pl.ds(0, _SORT_PAD)])
  pltpu.sync_copy(sz_hbm, sz_v.at[pl.ds(0, _SORT_PAD)])
  pltpu.sync_copy(si_hbm, si_v.at[pl.ds(0, _SORT_PAD)])
  pltpu.sync_copy(start_hbm.at[pl.ds(0, _NCELL_PAD)],
                  start_v.at[pl.ds(0, _NCELL_PAD)])
  pltpu.sync_copy(q_hbm.at[pl.ds(qbase * 4, _CHUNK * 4)],
                  q_v.at[pl.ds(0, _CHUNK * 4)])
  lanes = lax.iota(jnp.int32, 16)
  inf16 = jnp.full((16,), _INF, jnp.float32)
  neg16 = jnp.full((16,), -1, jnp.int32)

  def per_query(j, _):
    qvec = q_v[pl.ds(j * 4, 16)]
    qivec = jnp.clip(qvec.astype(jnp.int32), 0, 9)
    qb = qivec[0]
    qx = qvec[1]
    qy = qvec[2]
    qz = qvec[3]
    fx = qivec[1]
    fy = qivec[2]
    fz = qivec[3]
    z0 = jnp.maximum(fz - 1, 0)
    z1 = jnp.minimum(fz + 1, 9)

    def per_seg(t, cnt):
      dx = t // 3 - 1
      dy = t % 3 - 1
      cx = fx + dx
      cy = fy + dy
      ok = (cx >= 0) & (cx <= 9) & (cy >= 0) & (cy <= 9)
      cbase = ((qb * 10 + cx) * 10 + cy) * 10
      cb = jnp.clip(cbase, 0, _NCELL - 10)
      zl = lanes * 0
      s = plsc.load_gather(start_v, [zl + (cb + z0)])[0]
      e = plsc.load_gather(start_v, [zl + (cb + z1 + 1)])[0]
      e = jnp.where(ok, e, s)
      ng = (e - s + 15) // 16

      def per_group(g, cnt):
        p = s + g * 16
        rows = p + lanes
        rx = plsc.load_gather(sx_v, [rows])
        ry = plsc.load_gather(sy_v, [rows])
        rz = plsc.load_gather(sz_v, [rows])
        ridx = plsc.load_gather(si_v, [rows])
        ddx = rx - qx
        ddy = ry - qy
        ddz = rz - qz
        d2 = ddx * ddx + ddy * ddy + ddz * ddz
        m = ((p + lanes) < e) & (d2 <= _RADIUS2)
        cw = jnp.minimum(cnt, _CAP - 16)
        plsc.store_compressed(pend_d.at[pl.ds(cw, 16)], d2, mask=m)
        plsc.store_compressed(pend_i.at[pl.ds(cw, 16)], ridx, mask=m)
        return cnt + plsc.all_reduce_population_count(m)[0]

      return lax.fori_loop(0, ng, per_group, cnt)

    cnt = lax.fori_loop(0, 9, per_seg, jnp.int32(0))
    a_init = (inf16, neg16, inf16, neg16)

    def fast_path(args):
      def mg(gg, A):
        a0d, a0i, a1d, a1i = A
        base = gg * 16

        def do(A):
          a0d, a0i, a1d, a1i = A
          pd = pend_d[pl.ds(base, 16)]
          pi = pend_i[pl.ds(base, 16)]
          pd = jnp.where((base + lanes) < cnt, pd, _INF)
          pd, pi = plsc.sort_key_val(pd, pi)
          return _merge_top32(a0d, a0i, a1d, a1i, pd, pi)

        return lax.cond(base < cnt, do, lambda A: A, A)

      return lax.fori_loop(0, _CAP // 16, mg, args)

    def slow_path(args):
      def sg(gg, A):
        a0d, a0i, a1d, a1i = A
        p = gg * 16
        sl = pl.ds(p, 16)
        rx = sx_v[sl]
        ry = sy_v[sl]
        rz = sz_v[sl]
        ridx = si_v[sl]
        ddx = rx - qx
        ddy = ry - qy
        ddz = rz - qz
        d2 = ddx * ddx + ddy * ddy + ddz * ddz
        idx = ridx & 0xFFFFF
        bt = ridx >> 20
        m = ((d2 <= _RADIUS2) & (bt == qb) & (idx < _N_REF)
             & ((p + lanes) < _N_PAD))

        def do(A):
          a0d, a0i, a1d, a1i = A
          pd = jnp.where(m, d2, _INF)
          pd, pi = plsc.sort_key_val(pd, ridx)
          return _merge_top32(a0d, a0i, a1d, a1i, pd, pi)

        return lax.cond(jnp.any(m), do, lambda A: A, A)

      return lax.fori_loop(0, _NSG, sg, (inf16, neg16, inf16, neg16))

    a0d, a0i, a1d, a1i = lax.cond(cnt <= _CAP - 16, fast_path, slow_path,
                                  a_init)

    o0 = jnp.where(a0d <= _RADIUS2, a0i & 0xFFFFF, -1)
    o1 = jnp.where(a1d <= _RADIUS2, a1i & 0xFFFFF, -1)
    row_v[0:16] = o0
    row_v[16:32] = o1
    pltpu.sync_copy(row_v, out_hbm.at[pl.ds((qbase + j) * _K, _K)])
    return 0

  lax.fori_loop(0, _CHUNK, per_query, 0)


def kernel(ref_bxyz, query_bxyz):
  ref_pad = jnp.pad(ref_bxyz, ((0, _N_PAD - _N_REF), (0, 0)))
  q_pad = jnp.pad(query_bxyz, ((0, _N_PAD - _N_QUERY), (0, 0)))
  mesh = _mesh()
  cp = pltpu.CompilerParams(needs_layout_passes=False)

  cells, hist = pl.kernel(
      _hist_body,
      out_type=(jax.ShapeDtypeStruct((_N_PAD,), jnp.int32),
                jax.ShapeDtypeStruct((_NW * _NCELL_PAD,), jnp.int32)),
      mesh=mesh,
      compiler_params=cp,
      scratch_types=[
          pltpu.VMEM((_CHUNK * 4 + 16,), jnp.float32),
          pltpu.VMEM((_NCELL_PAD + 16,), jnp.int32),
          pltpu.VMEM((_CHUNK + 16,), jnp.int32),
      ],
  )(ref_pad.reshape(-1))

  sx, sy, sz, si, start = pl.kernel(
      _scatter_body,
      out_type=(jax.ShapeDtypeStruct((_SORT_PAD,), jnp.float32),
                jax.ShapeDtypeStruct((_SORT_PAD,), jnp.float32),
                jax.ShapeDtypeStruct((_SORT_PAD,), jnp.float32),
                jax.ShapeDtypeStruct((_SORT_PAD,), jnp.int32),
                jax.ShapeDtypeStruct((_NCELL_PAD * 8,), jnp.int32)),
      mesh=mesh,
      compiler_params=cp,
      scratch_types=[
          pltpu.VMEM((_CHUNK * 4 + 16,), jnp.float32),
          pltpu.VMEM((_CHUNK + 16,), jnp.int32),
          pltpu.VMEM((_NCELL_PAD,), jnp.int32),
          pltpu.VMEM((_NCELL_PAD + 16,), jnp.int32),
          pltpu.VMEM((_NCELL_PAD + 16,), jnp.int32),
          pltpu.VMEM((_NCELL_PAD + 16,), jnp.int32),
          pltpu.VMEM((1024,), jnp.float32),
          pltpu.VMEM((1024,), jnp.float32),
          pltpu.VMEM((1024,), jnp.float32),
          pltpu.VMEM((1024,), jnp.int32),
          pltpu.VMEM((1024 + 16,), jnp.int32),
          pltpu.VMEM((8, 128), jnp.int32),
          pltpu.SemaphoreType.DMA,
      ],
  )(ref_pad.reshape(-1), cells, hist)

  nbr = pl.kernel(
      _query_body,
      out_type=jax.ShapeDtypeStruct((_N_PAD * _K,), jnp.int32),
      mesh=mesh,
      compiler_params=cp,
      scratch_types=[
          pltpu.VMEM((_SORT_PAD + 16,), jnp.float32),
          pltpu.VMEM((_SORT_PAD + 16,), jnp.float32),
          pltpu.VMEM((_SORT_PAD + 16,), jnp.float32),
          pltpu.VMEM((_SORT_PAD + 16,), jnp.int32),
          pltpu.VMEM((_NCELL_PAD + 16,), jnp.int32),
          pltpu.VMEM((_CHUNK * 4 + 16,), jnp.float32),
          pltpu.VMEM((_K,), jnp.int32),
          pltpu.VMEM((_CAP + 16,), jnp.float32),
          pltpu.VMEM((_CAP + 16,), jnp.int32),
      ],
  )(sx, sy, sz, si, start, q_pad.reshape(-1))

  ref_idx = nbr[: _N_QUERY * _K]
  q_idx = jnp.repeat(jnp.arange(_N_QUERY, dtype=jnp.int32), _K)
  return jnp.stack([ref_idx, q_idx])
